# baseline (speedup 1.0000x reference)
import functools

import jax
import jax.numpy as jnp
from jax import lax
from jax.experimental import pallas as pl
from jax.experimental.pallas import tpu as pltpu
from jax.experimental.pallas import tpu_sc as plsc

VOCAB = 100
PAD = 128
EMB = 10
REP = 8
B = 4096
SEQ = 50

_info = plsc.get_sparse_core_info()
NC, NS = _info.num_cores, _info.num_subcores
NW = NC * NS
B_PER_W = B // NW
UNITS = B_PER_W // 2


def _gram_body(t_ref, b_ref, g_ref):
    t = t_ref[...]
    g = jnp.dot(t, t.T, preferred_element_type=jnp.float32) + b_ref[...]
    g = jnp.pad(g, ((0, 0), (0, PAD - VOCAB)))
    g_ref[...] = jnp.concatenate([g] * REP, axis=0)


def _compute_gram(table, bias):
    return pl.pallas_call(
        _gram_body,
        out_shape=jax.ShapeDtypeStruct((REP * VOCAB, PAD), jnp.float32),
    )(table, bias.reshape(1, VOCAB))


_sc_mesh = plsc.VectorSubcoreMesh(core_axis_name="c", subcore_axis_name="s")


@functools.partial(
    pl.kernel,
    out_type=jax.ShapeDtypeStruct((B, SEQ, VOCAB), jnp.float32),
    mesh=_sc_mesh,
    scratch_types=[
        pltpu.VMEM((B_PER_W, SEQ), jnp.int32),
        pltpu.VMEM((UNITS, 1, 2 * SEQ), jnp.int32),
        pltpu.VMEM((4, 2 * SEQ, PAD), jnp.float32),
        pltpu.VMEM((2, 2, SEQ, VOCAB), jnp.float32),
        pltpu.SemaphoreType.DMA,
        pltpu.SemaphoreType.DMA,
        pltpu.SemaphoreType.DMA,
        pltpu.SemaphoreType.DMA,
        pltpu.SemaphoreType.DMA,
        pltpu.SemaphoreType.DMA,
    ],
)
def _sc_gather(idx_hbm, g_hbm, out_hbm, idx_v, idxf_v, rows_v, outb_v,
               gsem0, gsem1, gsem2, gsem3, wsem0, wsem1):
    sid = lax.axis_index("s")
    wid = sid * NC + lax.axis_index("c")
    b0 = wid * B_PER_W
    gsems = (gsem0, gsem1, gsem2, gsem3)
    wsems = (wsem0, wsem1)

    pltpu.sync_copy(idx_hbm.at[pl.ds(b0, B_PER_W)], idx_v)

    rep_off = (wid % REP) * VOCAB

    @plsc.parallel_loop(0, UNITS, unroll=2)
    def _flatten(u):
        for h in range(2):
            src = 2 * u + h
            for (so, do) in ((0, 0), (16, 16), (32, 32), (34, 34)):
                v = idx_v[src, pl.ds(so, 16)]
                idxf_v[u, 0, pl.ds(h * SEQ + do, 16)] = v + rep_off

    def _start_gather(u):
        p = u % 4
        return pltpu.async_copy(
            g_hbm.at[idxf_v.at[u, 0]], rows_v.at[p], gsems[p])

    def _compact_unit(u):
        p = u % 4
        q = u % 2

        @plsc.parallel_loop(0, SEQ, unroll=2)
        def _compact(l):
            for h in range(2):
                for c in range(VOCAB // 16):
                    outb_v[q, h, l, pl.ds(c * 16, 16)] = (
                        rows_v[p, h * SEQ + l, pl.ds(c * 16, 16)])
                outb_v[q, h, l, pl.ds(VOCAB - 16, 16)] = (
                    rows_v[p, h * SEQ + l, pl.ds(VOCAB - 16, 16)])

    gathers = {}
    writes = {}
    for u in range(3):
        gathers[u] = _start_gather(u)
    for u in range(UNITS):
        gathers.pop(u).wait()
        if u + 3 < UNITS:
            gathers[u + 3] = _start_gather(u + 3)
        if u - 2 in writes:
            writes.pop(u - 2).wait()
        _compact_unit(u)
        q = u % 2
        writes[u] = pltpu.async_copy(
            outb_v.at[q], out_hbm.at[pl.ds(b0 + 2 * u, 2)], wsems[q])
    for u in sorted(writes):
        writes.pop(u).wait()


def kernel(input_vec, table, bias):
    g = _compute_gram(table, bias)
    return _sc_gather(input_vec.astype(jnp.int32), g)


# speedup vs baseline: 1.3824x; 1.3824x over previous
import functools

import jax
import jax.numpy as jnp
from jax import lax
from jax.experimental import pallas as pl
from jax.experimental.pallas import tpu as pltpu
from jax.experimental.pallas import tpu_sc as plsc

VOCAB = 100
PAD = 128
EMB = 10
REP = 8
B = 4096
SEQ = 50

_info = plsc.get_sparse_core_info()
NC, NS = _info.num_cores, _info.num_subcores
NW = NC * NS
B_PER_W = B // NW
UNITS = B_PER_W // 2


def _gram_body(t_ref, b_ref, g_ref):
    t = t_ref[...]
    g = jnp.dot(t, t.T, preferred_element_type=jnp.float32) + b_ref[...]
    g = jnp.pad(g, ((0, 0), (0, PAD - VOCAB)))
    g_ref[...] = jnp.concatenate([g] * REP, axis=0)


def _compute_gram(table, bias):
    return pl.pallas_call(
        _gram_body,
        out_shape=jax.ShapeDtypeStruct((REP * VOCAB, PAD), jnp.float32),
    )(table, bias.reshape(1, VOCAB))


_sc_mesh = plsc.VectorSubcoreMesh(core_axis_name="c", subcore_axis_name="s")


@functools.partial(
    pl.kernel,
    out_type=jax.ShapeDtypeStruct((B, SEQ, VOCAB), jnp.float32),
    mesh=_sc_mesh,
    scratch_types=[
        pltpu.VMEM((B_PER_W, SEQ), jnp.int32),
        pltpu.VMEM((UNITS, 1, 2 * SEQ), jnp.int32),
        pltpu.VMEM((4, 2 * SEQ, PAD), jnp.float32),
        pltpu.VMEM((2, 2, SEQ, VOCAB), jnp.float32),
        pltpu.VMEM_SHARED((REP * VOCAB, PAD), jnp.float32),
        pltpu.SemaphoreType.DMA,
        pltpu.SemaphoreType.DMA,
        pltpu.SemaphoreType.DMA,
        pltpu.SemaphoreType.DMA,
        pltpu.SemaphoreType.DMA,
        pltpu.SemaphoreType.DMA,
    ],
)
def _sc_gather(idx_hbm, g_hbm, out_hbm, idx_v, idxf_v, rows_v, outb_v, g_sh,
               gsem0, gsem1, gsem2, gsem3, wsem0, wsem1):
    sid = lax.axis_index("s")
    wid = sid * NC + lax.axis_index("c")
    b0 = wid * B_PER_W
    gsems = (gsem0, gsem1, gsem2, gsem3)
    wsems = (wsem0, wsem1)

    @pl.when(sid < 10)
    def _stage():
        pltpu.sync_copy(g_hbm.at[pl.ds(sid * 80, 80)],
                        rows_v.at[0, pl.ds(0, 80)])
        pltpu.sync_copy(rows_v.at[0, pl.ds(0, 80)],
                        g_sh.at[pl.ds(sid * 80, 80)])

    pltpu.sync_copy(idx_hbm.at[pl.ds(b0, B_PER_W)], idx_v)
    plsc.subcore_barrier()

    rep_off = (wid % REP) * VOCAB

    @plsc.parallel_loop(0, UNITS, unroll=2)
    def _flatten(u):
        for h in range(2):
            src = 2 * u + h
            for (so, do) in ((0, 0), (16, 16), (32, 32), (34, 34)):
                v = idx_v[src, pl.ds(so, 16)]
                idxf_v[u, 0, pl.ds(h * SEQ + do, 16)] = v + rep_off

    def _start_gather(u):
        p = u % 4
        return pltpu.async_copy(
            g_sh.at[idxf_v.at[u, 0]], rows_v.at[p], gsems[p])

    def _compact_unit(u):
        p = u % 4
        q = u % 2

        @plsc.parallel_loop(0, SEQ, unroll=2)
        def _compact(l):
            for h in range(2):
                for c in range(VOCAB // 16):
                    outb_v[q, h, l, pl.ds(c * 16, 16)] = (
                        rows_v[p, h * SEQ + l, pl.ds(c * 16, 16)])
                outb_v[q, h, l, pl.ds(VOCAB - 16, 16)] = (
                    rows_v[p, h * SEQ + l, pl.ds(VOCAB - 16, 16)])

    gathers = {}
    writes = {}
    for u in range(3):
        gathers[u] = _start_gather(u)
    for u in range(UNITS):
        gathers.pop(u).wait()
        if u + 3 < UNITS:
            gathers[u + 3] = _start_gather(u + 3)
        if u - 2 in writes:
            writes.pop(u - 2).wait()
        _compact_unit(u)
        q = u % 2
        writes[u] = pltpu.async_copy(
            outb_v.at[q], out_hbm.at[pl.ds(b0 + 2 * u, 2)], wsems[q])
    for u in sorted(writes):
        writes.pop(u).wait()


def kernel(input_vec, table, bias):
    g = _compute_gram(table, bias)
    return _sc_gather(input_vec.astype(jnp.int32), g)
